# baseline (speedup 1.0000x reference)
import jax
import jax.numpy as jnp
from jax import lax
from jax.experimental import pallas as pl
from jax.experimental.pallas import tpu as pltpu
from jax.experimental.pallas import tpu_sc as plsc

N_TOK = 10
D = 64
B = 1024
S = 200
SEQ_G = S - N_TOK
HALF = SEQ_G // 2

NC = 2
NS = 16
NW = NC * NS
BPW = B // NW
CH = 4
NCH = BPW // CH


def _soft_embed_sc(table, idx95, learned, out,
                   idx_v, buf0, buf1, sem0, sem1):
    wid = lax.axis_index("s") * NC + lax.axis_index("c")
    bufs = (buf0, buf1)
    sems = (sem0, sem1)

    pltpu.sync_copy(idx95.at[pl.ds(wid * BPW * 2, BPW * 2)], idx_v)

    for nb in range(2):
        for j in range(CH):
            pltpu.sync_copy(learned, bufs[nb].at[pl.ds(j * S, N_TOK)])

    def fetch(c, nb):
        dmas = []
        for j in range(CH * 2):
            dst = bufs[nb].at[pl.ds((j // 2) * S + N_TOK + (j % 2) * HALF, HALF)]
            src = table.at[idx_v.at[c * CH * 2 + j]]
            dmas.append(pltpu.async_copy(src, dst, sems[nb]))
        return dmas

    pending = fetch(0, 0)
    for c in range(NCH):
        nb = c % 2
        nxt = fetch(c + 1, 1 - nb) if c + 1 < NCH else None
        for d in pending:
            d.wait()
        b0 = wid * BPW + c * CH
        pltpu.sync_copy(bufs[nb], out.at[pl.ds(b0 * S, CH * S)])
        pending = nxt


def kernel(tokens, wte_weight, learned_embedding):
    idx95 = tokens[:, N_TOK:].reshape(B * 2, HALF)
    mesh = plsc.VectorSubcoreMesh(core_axis_name="c", subcore_axis_name="s")
    f = pl.kernel(
        _soft_embed_sc,
        mesh=mesh,
        compiler_params=pltpu.CompilerParams(use_tc_tiling_on_sc=False),
        out_type=jax.ShapeDtypeStruct((B * S, D), jnp.float32),
        scratch_types=[
            pltpu.VMEM((BPW * 2, HALF), jnp.int32),
            pltpu.VMEM((CH * S, D), jnp.float32),
            pltpu.VMEM((CH * S, D), jnp.float32),
            pltpu.SemaphoreType.DMA,
            pltpu.SemaphoreType.DMA,
        ],
    )
    out = f(wte_weight, idx95, learned_embedding)
    return out.reshape(B, S, D)


# speedup vs baseline: 1.0574x; 1.0574x over previous
import jax
import jax.numpy as jnp
from jax import lax
from jax.experimental import pallas as pl
from jax.experimental.pallas import tpu as pltpu
from jax.experimental.pallas import tpu_sc as plsc

N_TOK = 10
D = 64
DP = 128
B = 1024
S = 200
SEQ_G = S - N_TOK
HALF = SEQ_G // 2

NC = 2
NS = 16
NW = NC * NS
BPW = B // NW
CH = 2
NCH = BPW // CH


def _soft_embed_sc(table, idx95, learned, out,
                   idx_v0, idx_v1, buf0, buf1, sem0, sem1):
    wid = lax.axis_index("s") * NC + lax.axis_index("c")
    idxs = (idx_v0, idx_v1)
    bufs = (buf0, buf1)
    sems = (sem0, sem1)

    for nb in range(2):
        for j in range(CH):
            pltpu.sync_copy(learned, bufs[nb].at[pl.ds(j * S, N_TOK), pl.ds(0, D)])

    def fetch(c, nb):
        b0 = wid * BPW + c * CH
        pltpu.sync_copy(idx95.at[pl.ds(b0 * 2, CH * 2)], idxs[nb])
        dmas = []
        for j in range(CH * 2):
            dst = bufs[nb].at[pl.ds((j // 2) * S + N_TOK + (j % 2) * HALF, HALF)]
            src = table.at[idxs[nb].at[j]]
            dmas.append(pltpu.async_copy(src, dst, sems[nb]))
        return dmas

    pending = fetch(0, 0)
    for c in range(NCH):
        nb = c % 2
        nxt = fetch(c + 1, 1 - nb) if c + 1 < NCH else None
        for d in pending:
            d.wait()
        b0 = wid * BPW + c * CH
        for j in range(CH):
            pltpu.sync_copy(bufs[nb].at[pl.ds(j * S, S), pl.ds(0, D)],
                            out.at[b0 + j])
        pending = nxt


def kernel(tokens, wte_weight, learned_embedding):
    idx95 = tokens[:, N_TOK:].reshape(B * 2, HALF)
    wte_pad = jnp.pad(wte_weight, ((0, 0), (0, DP - D)))
    mesh = plsc.VectorSubcoreMesh(core_axis_name="c", subcore_axis_name="s")
    f = pl.kernel(
        _soft_embed_sc,
        mesh=mesh,
        compiler_params=pltpu.CompilerParams(use_tc_tiling_on_sc=False),
        out_type=jax.ShapeDtypeStruct((B, S, D), jnp.float32),
        scratch_types=[
            pltpu.VMEM((CH * 2, HALF), jnp.int32),
            pltpu.VMEM((CH * 2, HALF), jnp.int32),
            pltpu.VMEM((CH * S, DP), jnp.float32),
            pltpu.VMEM((CH * S, DP), jnp.float32),
            pltpu.SemaphoreType.DMA,
            pltpu.SemaphoreType.DMA,
        ],
    )
    return f(wte_pad, idx95, learned_embedding)
